# baseline (speedup 1.0000x reference)
import math

import jax
import jax.numpy as jnp
from jax.experimental import pallas as pl
from jax.experimental.pallas import tpu as pltpu

_VMEM_LIMIT = 64 * 1024 * 1024


def _fused_cnn_kernel(x_ref, a1_ref, b1_ref,
                      a2_ref, b2_ref, w1_ref, c1_ref, w2_ref, c2_ref,
                      o_ref, *, bt):
    f32 = jnp.float32
    bf16 = jnp.bfloat16
    xv = x_ref[...]
    z1 = jnp.zeros((1, bt, 32), bf16)
    xm1 = jnp.concatenate([z1, xv[:7, :, 96:128]], axis=0)
    xp4 = jnp.concatenate([xv[1:, :, 0:32], z1], axis=0)

    scats = {
        -1: jnp.concatenate([xm1, xv[:, :, 0:64]], axis=2),
        0: xv[:, :, 0:96],
        1: xv[:, :, 32:128],
        2: jnp.concatenate([xv[:, :, 64:128], xp4], axis=2),
    }
    p1 = []
    for mp in (0, 1):
        zmax = None
        for ph in (0, 1):
            scat = scats[2 * mp + ph - 1].reshape(8 * bt, 96)
            z = jnp.dot(scat, a1_ref[...], preferred_element_type=f32)
            zp = jnp.maximum(z[:, :512], z[:, 512:])
            zmax = zp if zmax is None else jnp.maximum(zmax, zp)
        p1.append(jnp.maximum(zmax + b1_ref[...], 0.0)
                  .astype(bf16).reshape(8, bt, 512))

    z2 = jnp.zeros((1, bt, 512), bf16)
    s2 = {
        -1: jnp.concatenate([z2, p1[1][:7]], axis=0),
        0: p1[0],
        1: p1[1],
        2: jnp.concatenate([p1[0][1:], z2], axis=0),
    }
    z2max = None
    for ph in (0, 1):
        z = None
        for kh in range(3):
            zk = jnp.dot(s2[ph + kh - 1].reshape(8 * bt, 512), a2_ref[kh],
                         preferred_element_type=f32)
            z = zk if z is None else z + zk
        zp = jnp.maximum(z[:, :512], z[:, 512:])
        z2max = zp if z2max is None else jnp.maximum(z2max, zp)
    p2 = jnp.maximum(z2max + b2_ref[...], 0.0)

    p2r = p2.reshape(8, bt, 512)
    acc = jnp.zeros((bt, 128), f32)
    for m2 in range(8):
        acc = acc + jnp.dot(p2r[m2], w1_ref[m2],
                            preferred_element_type=f32)
    h = jnp.maximum(acc + c1_ref[...], 0.0)
    o_ref[...] = jnp.dot(h, w2_ref[...], preferred_element_type=f32) + c2_ref[...]


def _band_matrices(conv1_w, conv2_w):
    hp = jax.lax.Precision.HIGHEST
    w1 = conv1_w.reshape(3, 3, 32)
    kk = jnp.arange(3)[:, None, None, None]
    jj = jnp.arange(32)[None, :, None, None]
    pp = jnp.arange(2)[None, None, :, None]
    nn = jnp.arange(16)[None, None, None, :]
    m1 = (jj == 2 * nn + pp + kk - 1).astype(jnp.float32)
    a1 = jnp.einsum('xkc,kjpn->xjpnc', w1, m1, precision=hp).reshape(96, 1024)

    w2 = conv2_w.reshape(3, 3, 32, 64)
    nn1 = jnp.arange(16)[None, :, None, None]
    pp2 = jnp.arange(2)[None, None, :, None]
    nn2 = jnp.arange(8)[None, None, None, :]
    m2 = (nn1 == 2 * nn2 + pp2 + jnp.arange(3)[:, None, None, None] - 1
          ).astype(jnp.float32)
    a2 = jnp.einsum('xkio,knpq->xnipqo', w2, m2,
                    precision=hp).reshape(3, 512, 1024)
    return a1, a2


def kernel(conv1_w, conv1_b, conv2_w, conv2_b, fc1_w, fc1_b, fc2_w, fc2_b, x_nchw):
    B = x_nchw.shape[0]
    bt = math.gcd(B, 128)
    x = x_nchw.reshape(B, 8, 128).transpose(1, 0, 2).astype(jnp.bfloat16)

    a1, a2 = _band_matrices(conv1_w, conv2_w)
    a1 = a1.astype(jnp.bfloat16)
    a2 = a2.astype(jnp.bfloat16)
    b1t = jnp.tile(conv1_b.reshape(32), (16,)).reshape(1, 512)
    b2t = jnp.tile(conv2_b.reshape(64), (8,)).reshape(1, 512)
    w1r = fc1_w.reshape(8, 512, 128)

    out = pl.pallas_call(
        lambda *refs: _fused_cnn_kernel(*refs, bt=bt),
        out_shape=jax.ShapeDtypeStruct((B, 128), jnp.float32),
        grid_spec=pltpu.PrefetchScalarGridSpec(
            num_scalar_prefetch=0,
            grid=(B // bt,),
            in_specs=[
                pl.BlockSpec((8, bt, 128), lambda i: (0, i, 0)),
                pl.BlockSpec((96, 1024), lambda i: (0, 0)),
                pl.BlockSpec((1, 512), lambda i: (0, 0)),
                pl.BlockSpec((3, 512, 1024), lambda i: (0, 0, 0)),
                pl.BlockSpec((1, 512), lambda i: (0, 0)),
                pl.BlockSpec((8, 512, 128), lambda i: (0, 0, 0)),
                pl.BlockSpec((1, 128), lambda i: (0, 0)),
                pl.BlockSpec((128, 128), lambda i: (0, 0)),
                pl.BlockSpec((1, 128), lambda i: (0, 0)),
            ],
            out_specs=pl.BlockSpec((bt, 128), lambda i: (i, 0)),
        ),
        compiler_params=pltpu.CompilerParams(
            dimension_semantics=("parallel",), vmem_limit_bytes=_VMEM_LIMIT),
    )(x, a1, b1t, a2, b2t, w1r, fc1_b, fc2_w, fc2_b)
    return out[:, :10]


# speedup vs baseline: 1.0346x; 1.0346x over previous
import math

import jax
import jax.numpy as jnp
from jax.experimental import pallas as pl
from jax.experimental.pallas import tpu as pltpu

_VMEM_LIMIT = 64 * 1024 * 1024


def _fused_cnn_kernel(x_ref, a1_ref, b1_ref,
                      a2_ref, b2_ref, w1_ref, c1_ref, w2_ref, c2_ref,
                      o_ref, *, bt):
    f32 = jnp.float32
    xv = x_ref[...]
    z1 = jnp.zeros((1, bt, 32), f32)
    xm1 = jnp.concatenate([z1, xv[:7, :, 96:128]], axis=0)
    xp4 = jnp.concatenate([xv[1:, :, 0:32], z1], axis=0)

    scats = {
        -1: jnp.concatenate([xm1, xv[:, :, 0:64]], axis=2),
        0: xv[:, :, 0:96],
        1: xv[:, :, 32:128],
        2: jnp.concatenate([xv[:, :, 64:128], xp4], axis=2),
    }
    p1 = []
    for mp in (0, 1):
        zmax = None
        for ph in (0, 1):
            scat = scats[2 * mp + ph - 1].reshape(8 * bt, 96)
            z = jnp.dot(scat, a1_ref[...], preferred_element_type=f32)
            zp = jnp.maximum(z[:, :512], z[:, 512:])
            zmax = zp if zmax is None else jnp.maximum(zmax, zp)
        p1.append(jnp.maximum(zmax + b1_ref[...], 0.0).reshape(8, bt, 512))

    z2 = jnp.zeros((1, bt, 512), f32)
    s2 = {
        -1: jnp.concatenate([z2, p1[1][:7]], axis=0),
        0: p1[0],
        1: p1[1],
        2: jnp.concatenate([p1[0][1:], z2], axis=0),
    }
    z2max = None
    for ph in (0, 1):
        z = None
        for kh in range(3):
            zk = jnp.dot(s2[ph + kh - 1].reshape(8 * bt, 512), a2_ref[kh],
                         preferred_element_type=f32)
            z = zk if z is None else z + zk
        zp = jnp.maximum(z[:, :512], z[:, 512:])
        z2max = zp if z2max is None else jnp.maximum(z2max, zp)
    p2 = jnp.maximum(z2max + b2_ref[...], 0.0)

    p2r = p2.reshape(8, bt, 512)
    acc = jnp.zeros((bt, 128), f32)
    for m2 in range(8):
        acc = acc + jnp.dot(p2r[m2], w1_ref[m2],
                            preferred_element_type=f32)
    h = jnp.maximum(acc + c1_ref[...], 0.0)
    o_ref[...] = jnp.dot(h, w2_ref[...], preferred_element_type=f32) + c2_ref[...]


def _band_matrices(conv1_w, conv2_w):
    hp = jax.lax.Precision.HIGHEST
    w1 = conv1_w.reshape(3, 3, 32)
    kk = jnp.arange(3)[:, None, None, None]
    jj = jnp.arange(32)[None, :, None, None]
    pp = jnp.arange(2)[None, None, :, None]
    nn = jnp.arange(16)[None, None, None, :]
    m1 = (jj == 2 * nn + pp + kk - 1).astype(jnp.float32)
    a1 = jnp.einsum('xkc,kjpn->xjpnc', w1, m1, precision=hp).reshape(96, 1024)

    w2 = conv2_w.reshape(3, 3, 32, 64)
    nn1 = jnp.arange(16)[None, :, None, None]
    pp2 = jnp.arange(2)[None, None, :, None]
    nn2 = jnp.arange(8)[None, None, None, :]
    m2 = (nn1 == 2 * nn2 + pp2 + jnp.arange(3)[:, None, None, None] - 1
          ).astype(jnp.float32)
    a2 = jnp.einsum('xkio,knpq->xnipqo', w2, m2,
                    precision=hp).reshape(3, 512, 1024)
    return a1, a2


def kernel(conv1_w, conv1_b, conv2_w, conv2_b, fc1_w, fc1_b, fc2_w, fc2_b, x_nchw):
    B = x_nchw.shape[0]
    bt = math.gcd(B, 256)
    x = x_nchw.reshape(B, 8, 128).transpose(1, 0, 2)

    a1, a2 = _band_matrices(conv1_w, conv2_w)
    b1t = jnp.tile(conv1_b.reshape(32), (16,)).reshape(1, 512)
    b2t = jnp.tile(conv2_b.reshape(64), (8,)).reshape(1, 512)
    w1r = fc1_w.reshape(8, 512, 128)

    out = pl.pallas_call(
        lambda *refs: _fused_cnn_kernel(*refs, bt=bt),
        out_shape=jax.ShapeDtypeStruct((B, 128), jnp.float32),
        grid_spec=pltpu.PrefetchScalarGridSpec(
            num_scalar_prefetch=0,
            grid=(B // bt,),
            in_specs=[
                pl.BlockSpec((8, bt, 128), lambda i: (0, i, 0)),
                pl.BlockSpec((96, 1024), lambda i: (0, 0)),
                pl.BlockSpec((1, 512), lambda i: (0, 0)),
                pl.BlockSpec((3, 512, 1024), lambda i: (0, 0, 0)),
                pl.BlockSpec((1, 512), lambda i: (0, 0)),
                pl.BlockSpec((8, 512, 128), lambda i: (0, 0, 0)),
                pl.BlockSpec((1, 128), lambda i: (0, 0)),
                pl.BlockSpec((128, 128), lambda i: (0, 0)),
                pl.BlockSpec((1, 128), lambda i: (0, 0)),
            ],
            out_specs=pl.BlockSpec((bt, 128), lambda i: (i, 0)),
        ),
        compiler_params=pltpu.CompilerParams(
            dimension_semantics=("parallel",), vmem_limit_bytes=_VMEM_LIMIT),
    )(x, a1, b1t, a2, b2t, w1r, fc1_b, fc2_w, fc2_b)
    return out[:, :10]


# speedup vs baseline: 1.0416x; 1.0068x over previous
import math

import jax
import jax.numpy as jnp
from jax.experimental import pallas as pl
from jax.experimental.pallas import tpu as pltpu

_VMEM_LIMIT = 64 * 1024 * 1024


def _fused_cnn_kernel(x_ref, a1_ref, b1_ref,
                      a2_ref, b2_ref, w1_ref, c1_ref, w2_ref, c2_ref,
                      o_ref, *, bt):
    f32 = jnp.float32
    xv = jnp.transpose(x_ref[...], (1, 0, 2))
    z1 = jnp.zeros((1, bt, 32), f32)
    xm1 = jnp.concatenate([z1, xv[:7, :, 96:128]], axis=0)
    xp4 = jnp.concatenate([xv[1:, :, 0:32], z1], axis=0)

    scats = {
        -1: jnp.concatenate([xm1, xv[:, :, 0:64]], axis=2),
        0: xv[:, :, 0:96],
        1: xv[:, :, 32:128],
        2: jnp.concatenate([xv[:, :, 64:128], xp4], axis=2),
    }
    p1 = []
    for mp in (0, 1):
        zmax = None
        for ph in (0, 1):
            scat = scats[2 * mp + ph - 1].reshape(8 * bt, 96)
            z = jnp.dot(scat, a1_ref[...], preferred_element_type=f32)
            zp = jnp.maximum(z[:, :512], z[:, 512:])
            zmax = zp if zmax is None else jnp.maximum(zmax, zp)
        p1.append(jnp.maximum(zmax + b1_ref[...], 0.0).reshape(8, bt, 512))

    z2 = jnp.zeros((1, bt, 512), f32)
    s2 = {
        -1: jnp.concatenate([z2, p1[1][:7]], axis=0),
        0: p1[0],
        1: p1[1],
        2: jnp.concatenate([p1[0][1:], z2], axis=0),
    }
    z2max = None
    for ph in (0, 1):
        z = None
        for kh in range(3):
            zk = jnp.dot(s2[ph + kh - 1].reshape(8 * bt, 512), a2_ref[kh],
                         preferred_element_type=f32)
            z = zk if z is None else z + zk
        zp = jnp.maximum(z[:, :512], z[:, 512:])
        z2max = zp if z2max is None else jnp.maximum(z2max, zp)
    p2 = jnp.maximum(z2max + b2_ref[...], 0.0)

    p2r = p2.reshape(8, bt, 512)
    acc = jnp.zeros((bt, 128), f32)
    for m2 in range(8):
        acc = acc + jnp.dot(p2r[m2], w1_ref[m2],
                            preferred_element_type=f32)
    h = jnp.maximum(acc + c1_ref[...], 0.0)
    o_ref[...] = jnp.dot(h, w2_ref[...], preferred_element_type=f32) + c2_ref[...]


def _band_matrices(conv1_w, conv2_w):
    hp = jax.lax.Precision.HIGHEST
    w1 = conv1_w.reshape(3, 3, 32)
    kk = jnp.arange(3)[:, None, None, None]
    jj = jnp.arange(32)[None, :, None, None]
    pp = jnp.arange(2)[None, None, :, None]
    nn = jnp.arange(16)[None, None, None, :]
    m1 = (jj == 2 * nn + pp + kk - 1).astype(jnp.float32)
    a1 = jnp.einsum('xkc,kjpn->xjpnc', w1, m1, precision=hp).reshape(96, 1024)

    w2 = conv2_w.reshape(3, 3, 32, 64)
    nn1 = jnp.arange(16)[None, :, None, None]
    pp2 = jnp.arange(2)[None, None, :, None]
    nn2 = jnp.arange(8)[None, None, None, :]
    m2 = (nn1 == 2 * nn2 + pp2 + jnp.arange(3)[:, None, None, None] - 1
          ).astype(jnp.float32)
    a2 = jnp.einsum('xkio,knpq->xnipqo', w2, m2,
                    precision=hp).reshape(3, 512, 1024)
    return a1, a2


def kernel(conv1_w, conv1_b, conv2_w, conv2_b, fc1_w, fc1_b, fc2_w, fc2_b, x_nchw):
    B = x_nchw.shape[0]
    bt = math.gcd(B, 256)
    x = x_nchw.reshape(B, 8, 128)

    a1, a2 = _band_matrices(conv1_w, conv2_w)
    b1t = jnp.tile(conv1_b.reshape(32), (16,)).reshape(1, 512)
    b2t = jnp.tile(conv2_b.reshape(64), (8,)).reshape(1, 512)
    w1r = fc1_w.reshape(8, 512, 128)

    out = pl.pallas_call(
        lambda *refs: _fused_cnn_kernel(*refs, bt=bt),
        out_shape=jax.ShapeDtypeStruct((B, 128), jnp.float32),
        grid_spec=pltpu.PrefetchScalarGridSpec(
            num_scalar_prefetch=0,
            grid=(B // bt,),
            in_specs=[
                pl.BlockSpec((bt, 8, 128), lambda i: (i, 0, 0)),
                pl.BlockSpec((96, 1024), lambda i: (0, 0)),
                pl.BlockSpec((1, 512), lambda i: (0, 0)),
                pl.BlockSpec((3, 512, 1024), lambda i: (0, 0, 0)),
                pl.BlockSpec((1, 512), lambda i: (0, 0)),
                pl.BlockSpec((8, 512, 128), lambda i: (0, 0, 0)),
                pl.BlockSpec((1, 128), lambda i: (0, 0)),
                pl.BlockSpec((128, 128), lambda i: (0, 0)),
                pl.BlockSpec((1, 128), lambda i: (0, 0)),
            ],
            out_specs=pl.BlockSpec((bt, 128), lambda i: (i, 0)),
        ),
        compiler_params=pltpu.CompilerParams(
            dimension_semantics=("parallel",), vmem_limit_bytes=_VMEM_LIMIT),
    )(x, a1, b1t, a2, b2t, w1r, fc1_b, fc2_w, fc2_b)
    return out[:, :10]
